# Initial kernel scaffold; baseline (speedup 1.0000x reference)
import functools

import jax
import jax.numpy as jnp
from jax import lax
from jax.experimental import pallas as pl
from jax.experimental.pallas import tpu as pltpu
from jax.experimental.pallas import tpu_sc as plsc

N = 100000
E = 1600000
NH = 34
XD = 2
D = 40
HALF = 50000
ROWS_SH = 50016
TRASH = 50000
TILES = 16
EPT = E // TILES
CHUNK = 128
NFULL = EPT // CHUNK
REM = EPT - NFULL * CHUNK
ZROWS = ROWS_SH // TILES

BLK = 5000


def _pre_body(x_ref, we1, be1, we2, be2, wm1, bm1, wm2, bm2, h_ref, msg_ref):
    x = x_ref[...]
    h = jnp.maximum(jnp.dot(x, we1[...], preferred_element_type=jnp.float32)
                    + be1[...], 0.0)
    h = jnp.dot(h, we2[...], preferred_element_type=jnp.float32) + be2[...]
    h_ref[...] = h
    m = jnp.maximum(jnp.dot(h, wm1[...], preferred_element_type=jnp.float32)
                    + bm1[...], 0.0)
    m = jnp.dot(m, wm2[...], preferred_element_type=jnp.float32) + bm2[...]
    msg_ref[...] = jnp.concatenate(
        [m, jnp.ones((BLK, 1), jnp.float32), jnp.zeros((BLK, D - NH - 1), jnp.float32)],
        axis=1)


def _full(shape):
    return pl.BlockSpec(shape, lambda i: (0, 0))


_pre_call = pl.pallas_call(
    _pre_body,
    grid=(N // BLK,),
    in_specs=[
        pl.BlockSpec((BLK, XD), lambda i: (i, 0)),
        _full((XD, NH)), _full((1, NH)), _full((NH, NH)), _full((1, NH)),
        _full((NH, NH)), _full((1, NH)), _full((NH, NH)), _full((1, NH)),
    ],
    out_specs=[
        pl.BlockSpec((BLK, NH), lambda i: (i, 0)),
        pl.BlockSpec((BLK, D), lambda i: (i, 0)),
    ],
    out_shape=[
        jax.ShapeDtypeStruct((N, NH), jnp.float32),
        jax.ShapeDtypeStruct((N, D), jnp.float32),
    ],
)


_mesh = plsc.VectorSubcoreMesh(core_axis_name="c", subcore_axis_name="s")


@functools.partial(
    pl.kernel,
    mesh=_mesh,
    out_type=jax.ShapeDtypeStruct((2, ROWS_SH, D), jnp.float32),
    scratch_types=[
        pltpu.VMEM((CHUNK,), jnp.int32),
        pltpu.VMEM((CHUNK,), jnp.int32),
        pltpu.VMEM((CHUNK,), jnp.int32),
        pltpu.VMEM((CHUNK, D), jnp.float32),
        pltpu.VMEM_SHARED((ROWS_SH, D), jnp.float32),
        pltpu.SemaphoreType.DMA,
    ],
)
def _edge_pass(ei_hbm, msg_hbm, zeros_hbm, out_hbm,
               snd_v, rcv_v, ridx_v, rows_v, aggr_sh, sem):
    c = lax.axis_index("c")
    s = lax.axis_index("s")
    lo = c * HALF

    pltpu.sync_copy(zeros_hbm, aggr_sh.at[pl.ds(s * ZROWS, ZROWS)])
    plsc.subcore_barrier()

    base = s * EPT

    def do_chunk(off, first_valid):
        pltpu.sync_copy(ei_hbm.at[0, pl.ds(off, CHUNK)], snd_v)
        pltpu.sync_copy(ei_hbm.at[1, pl.ds(off, CHUNK)], rcv_v)
        pltpu.async_copy(msg_hbm.at[snd_v], rows_v, sem).wait()
        for j in range(CHUNK // 16):
            r = rcv_v[pl.ds(j * 16, 16)]
            ok = (r >= lo) & (r < lo + HALF)
            if first_valid is not None:
                lane = lax.iota(jnp.int32, 16) + (j * 16)
                ok = ok & (lane >= first_valid)
            ridx_v[pl.ds(j * 16, 16)] = jnp.where(ok, r - lo, TRASH)
        pltpu.sync_copy(rows_v, aggr_sh.at[ridx_v], add=True)

    def body(k, carry):
        do_chunk(base + k * CHUNK, None)
        return carry

    lax.fori_loop(0, NFULL, body, 0)
    do_chunk(base + EPT - CHUNK, CHUNK - REM)

    plsc.subcore_barrier()
    pltpu.sync_copy(aggr_sh.at[pl.ds(s * ZROWS, ZROWS)],
                    out_hbm.at[c, pl.ds(s * ZROWS, ZROWS)])


def _post_body(h_ref, a_ref, wn1, bn1, wn2, bn2, lns, lnb,
               wd1, bd1, wd2, bd2, wp1, bp1, wp2, bp2, wp3, bp3, out_ref):
    h = h_ref[...]
    a = a_ref[0]
    aggr = a[:, :NH]
    deg = a[:, NH:NH + 1]
    mean = aggr / jnp.maximum(deg, 1.0)
    u = jnp.concatenate([h, mean], axis=1)
    t = jnp.maximum(jnp.dot(u, wn1[...], preferred_element_type=jnp.float32)
                    + bn1[...], 0.0)
    t = jnp.dot(t, wn2[...], preferred_element_type=jnp.float32) + bn2[...]
    mu = jnp.mean(t, axis=1, keepdims=True)
    var = jnp.mean((t - mu) * (t - mu), axis=1, keepdims=True)
    t = (t - mu) * lax.rsqrt(var + 1e-5) * lns[...] + lnb[...]
    t = jnp.maximum(jnp.dot(t, wd1[...], preferred_element_type=jnp.float32)
                    + bd1[...], 0.0)
    t = jnp.dot(t, wd2[...], preferred_element_type=jnp.float32) + bd2[...]
    p = jnp.maximum(jnp.dot(t, wp1[...], preferred_element_type=jnp.float32)
                    + bp1[...], 0.0)
    p = jnp.maximum(jnp.dot(p, wp2[...], preferred_element_type=jnp.float32)
                    + bp2[...], 0.0)
    logits = jnp.dot(p, wp3[...], preferred_element_type=jnp.float32) + bp3[...]
    mx = jnp.max(logits, axis=1, keepdims=True)
    e = jnp.exp(logits - mx)
    out_ref[...] = e / jnp.sum(e, axis=1, keepdims=True)


_post_call = pl.pallas_call(
    _post_body,
    grid=(N // BLK,),
    in_specs=[
        pl.BlockSpec((BLK, NH), lambda i: (i, 0)),
        pl.BlockSpec((1, BLK, D), lambda i: (i // (HALF // BLK), i % (HALF // BLK), 0)),
        _full((2 * NH, NH)), _full((1, NH)), _full((NH, NH)), _full((1, NH)),
        _full((1, NH)), _full((1, NH)),
        _full((NH, NH)), _full((1, NH)), _full((NH, NH)), _full((1, NH)),
        _full((NH, NH)), _full((1, NH)), _full((NH, NH)), _full((1, NH)),
        _full((NH, 2)), _full((1, 2)),
    ],
    out_specs=pl.BlockSpec((BLK, 2), lambda i: (i, 0)),
    out_shape=jax.ShapeDtypeStruct((N, 2), jnp.float32),
)


def kernel(x, edge_index, W_enc1, b_enc1, W_enc2, b_enc2, W_msg1, b_msg1,
           W_msg2, b_msg2, W_nod1, b_nod1, W_nod2, b_nod2, ln_scale, ln_bias,
           W_dec1, b_dec1, W_dec2, b_dec2, W_p1, b_p1, W_p2, b_p2, W_p3, b_p3):
    r = lambda b: b.reshape(1, -1)
    h, msgpad = _pre_call(x, W_enc1, r(b_enc1), W_enc2, r(b_enc2),
                          W_msg1, r(b_msg1), W_msg2, r(b_msg2))
    zeros = jnp.zeros((ZROWS, D), jnp.float32)
    aggr_raw = _edge_pass(edge_index, msgpad, zeros)
    return _post_call(h, aggr_raw, W_nod1, r(b_nod1), W_nod2, r(b_nod2),
                      r(ln_scale), r(ln_bias), W_dec1, r(b_dec1),
                      W_dec2, r(b_dec2), W_p1, r(b_p1), W_p2, r(b_p2),
                      W_p3, r(b_p3))


# speedup vs baseline: 4.5300x; 4.5300x over previous
import functools

import jax
import jax.numpy as jnp
from jax import lax
from jax.experimental import pallas as pl
from jax.experimental.pallas import tpu as pltpu
from jax.experimental.pallas import tpu_sc as plsc

N = 100000
E = 1600000
NH = 34
XD = 2
D = 40
HALF = 50000
ROWS_SH = 50048
TRASH = 50000
TILES = 16
EPT = E // TILES
CHUNK = 128
NFULL = EPT // CHUNK
REM = EPT - NFULL * CHUNK
ZROWS = ROWS_SH // TILES

BLK = 5000


def _pre_body(x_ref, we1, be1, we2, be2, wm1, bm1, wm2, bm2, h_ref, msg_ref):
    x = x_ref[...]
    h = jnp.maximum(jnp.dot(x, we1[...], preferred_element_type=jnp.float32)
                    + be1[...], 0.0)
    h = jnp.dot(h, we2[...], preferred_element_type=jnp.float32) + be2[...]
    h_ref[...] = h
    m = jnp.maximum(jnp.dot(h, wm1[...], preferred_element_type=jnp.float32)
                    + bm1[...], 0.0)
    m = jnp.dot(m, wm2[...], preferred_element_type=jnp.float32) + bm2[...]
    msg_ref[...] = jnp.concatenate(
        [m, jnp.ones((BLK, 1), jnp.float32), jnp.zeros((BLK, D - NH - 1), jnp.float32)],
        axis=1)


def _full(shape):
    return pl.BlockSpec(shape, lambda i: (0, 0))


_pre_call = pl.pallas_call(
    _pre_body,
    grid=(N // BLK,),
    in_specs=[
        pl.BlockSpec((BLK, XD), lambda i: (i, 0)),
        _full((XD, NH)), _full((1, NH)), _full((NH, NH)), _full((1, NH)),
        _full((NH, NH)), _full((1, NH)), _full((NH, NH)), _full((1, NH)),
    ],
    out_specs=[
        pl.BlockSpec((BLK, NH), lambda i: (i, 0)),
        pl.BlockSpec((BLK, D), lambda i: (i, 0)),
    ],
    out_shape=[
        jax.ShapeDtypeStruct((N, NH), jnp.float32),
        jax.ShapeDtypeStruct((N, D), jnp.float32),
    ],
)


_mesh = plsc.VectorSubcoreMesh(core_axis_name="c", subcore_axis_name="s")


@functools.partial(
    pl.kernel,
    mesh=_mesh,
    out_type=jax.ShapeDtypeStruct((2, ROWS_SH, D), jnp.float32),
    scratch_types=[
        pltpu.VMEM((CHUNK,), jnp.int32),
        pltpu.VMEM((CHUNK,), jnp.int32),
        pltpu.VMEM((CHUNK,), jnp.int32),
        pltpu.VMEM((CHUNK, D), jnp.float32),
        pltpu.VMEM_SHARED((ROWS_SH, D), jnp.float32),
        pltpu.SemaphoreType.DMA,
    ],
    compiler_params=pltpu.CompilerParams(use_tc_tiling_on_sc=False),
)
def _edge_pass(ei_hbm, msg_hbm, zeros_hbm, out_hbm,
               snd_v, rcv_v, ridx_v, rows_v, aggr_sh, sem):
    c = lax.axis_index("c")
    s = lax.axis_index("s")
    lo = c * HALF

    pltpu.sync_copy(zeros_hbm, aggr_sh.at[pl.ds(s * ZROWS, ZROWS)])
    plsc.subcore_barrier()

    base = s * EPT

    def do_chunk(off, first_valid):
        pltpu.sync_copy(ei_hbm.at[pl.ds(off, CHUNK)], snd_v)
        pltpu.sync_copy(ei_hbm.at[pl.ds(E + off, CHUNK)], rcv_v)
        pltpu.async_copy(msg_hbm.at[snd_v], rows_v, sem).wait()
        for j in range(CHUNK // 16):
            r = rcv_v[pl.ds(j * 16, 16)]
            ok = (r >= lo) & (r < lo + HALF)
            if first_valid is not None:
                lane = lax.iota(jnp.int32, 16) + (j * 16)
                ok = ok & (lane >= first_valid)
            ridx_v[pl.ds(j * 16, 16)] = jnp.where(ok, r - lo, TRASH)
        pltpu.sync_copy(rows_v, aggr_sh.at[ridx_v], add=True)

    def body(k, carry):
        do_chunk(base + k * CHUNK, None)
        return carry

    lax.fori_loop(0, NFULL, body, 0)
    do_chunk(base + EPT - CHUNK, CHUNK - REM)

    plsc.subcore_barrier()
    pltpu.sync_copy(aggr_sh.at[pl.ds(s * ZROWS, ZROWS)],
                    out_hbm.at[c, pl.ds(s * ZROWS, ZROWS)])


def _post_body(h_ref, a_ref, wn1, bn1, wn2, bn2, lns, lnb,
               wd1, bd1, wd2, bd2, wp1, bp1, wp2, bp2, wp3, bp3, out_ref):
    h = h_ref[...]
    a = a_ref[0]
    aggr = a[:, :NH]
    deg = a[:, NH:NH + 1]
    mean = aggr / jnp.maximum(deg, 1.0)
    u = jnp.concatenate([h, mean], axis=1)
    t = jnp.maximum(jnp.dot(u, wn1[...], preferred_element_type=jnp.float32)
                    + bn1[...], 0.0)
    t = jnp.dot(t, wn2[...], preferred_element_type=jnp.float32) + bn2[...]
    mu = jnp.mean(t, axis=1, keepdims=True)
    var = jnp.mean((t - mu) * (t - mu), axis=1, keepdims=True)
    t = (t - mu) * lax.rsqrt(var + 1e-5) * lns[...] + lnb[...]
    t = jnp.maximum(jnp.dot(t, wd1[...], preferred_element_type=jnp.float32)
                    + bd1[...], 0.0)
    t = jnp.dot(t, wd2[...], preferred_element_type=jnp.float32) + bd2[...]
    p = jnp.maximum(jnp.dot(t, wp1[...], preferred_element_type=jnp.float32)
                    + bp1[...], 0.0)
    p = jnp.maximum(jnp.dot(p, wp2[...], preferred_element_type=jnp.float32)
                    + bp2[...], 0.0)
    logits = jnp.dot(p, wp3[...], preferred_element_type=jnp.float32) + bp3[...]
    mx = jnp.max(logits, axis=1, keepdims=True)
    e = jnp.exp(logits - mx)
    out_ref[...] = e / jnp.sum(e, axis=1, keepdims=True)


_post_call = pl.pallas_call(
    _post_body,
    grid=(N // BLK,),
    in_specs=[
        pl.BlockSpec((BLK, NH), lambda i: (i, 0)),
        pl.BlockSpec((1, BLK, D), lambda i: (i // (HALF // BLK), i % (HALF // BLK), 0)),
        _full((2 * NH, NH)), _full((1, NH)), _full((NH, NH)), _full((1, NH)),
        _full((1, NH)), _full((1, NH)),
        _full((NH, NH)), _full((1, NH)), _full((NH, NH)), _full((1, NH)),
        _full((NH, NH)), _full((1, NH)), _full((NH, NH)), _full((1, NH)),
        _full((NH, 2)), _full((1, 2)),
    ],
    out_specs=pl.BlockSpec((BLK, 2), lambda i: (i, 0)),
    out_shape=jax.ShapeDtypeStruct((N, 2), jnp.float32),
)


def kernel(x, edge_index, W_enc1, b_enc1, W_enc2, b_enc2, W_msg1, b_msg1,
           W_msg2, b_msg2, W_nod1, b_nod1, W_nod2, b_nod2, ln_scale, ln_bias,
           W_dec1, b_dec1, W_dec2, b_dec2, W_p1, b_p1, W_p2, b_p2, W_p3, b_p3):
    r = lambda b: b.reshape(1, -1)
    h, msgpad = _pre_call(x, W_enc1, r(b_enc1), W_enc2, r(b_enc2),
                          W_msg1, r(b_msg1), W_msg2, r(b_msg2))
    zeros = jnp.zeros((ZROWS, D), jnp.float32)
    aggr_raw = _edge_pass(edge_index.reshape(-1), msgpad, zeros)
    return _post_call(h, aggr_raw, W_nod1, r(b_nod1), W_nod2, r(b_nod2),
                      r(ln_scale), r(ln_bias), W_dec1, r(b_dec1),
                      W_dec2, r(b_dec2), W_p1, r(b_p1), W_p2, r(b_p2),
                      W_p3, r(b_p3))
